# initial kernel scaffold (unmeasured)
import jax
import jax.numpy as jnp
from jax import lax
from jax.experimental import pallas as pl
from jax.experimental.pallas import tpu as pltpu

HALF = 2048
D = 2048
NCHUNK = 8
CH = HALF // NCHUNK


def kernel(partial, gamma):
    p = partial.reshape(2 * HALF, D)
    g = gamma.reshape(1, D)

    def body(p_ref, g_ref, out_ref, local_ref, comm_ref,
             copy_sem, send_sem, recv_sem):
        my_x = lax.axis_index("x")
        my_y = lax.axis_index("y")
        my_z = lax.axis_index("z")
        peer = (my_x, 1 - my_y, my_z)

        my_start = my_y * HALF
        peer_start = (1 - my_y) * HALF

        barrier_sem = pltpu.get_barrier_semaphore()
        pl.semaphore_signal(
            barrier_sem, inc=1,
            device_id=peer, device_id_type=pl.DeviceIdType.MESH,
        )
        pl.semaphore_wait(barrier_sem, 1)

        local_copy = pltpu.make_async_copy(
            p_ref.at[pl.ds(my_start, HALF), :], local_ref, copy_sem,
        )
        local_copy.start()

        rdma = pltpu.make_async_remote_copy(
            src_ref=p_ref.at[pl.ds(peer_start, HALF), :],
            dst_ref=comm_ref,
            send_sem=send_sem,
            recv_sem=recv_sem,
            device_id=peer,
            device_id_type=pl.DeviceIdType.MESH,
        )
        rdma.start()

        local_copy.wait()
        rdma.wait()

        for k in range(NCHUNK):
            rows = pl.ds(k * CH, CH)
            y = local_ref[rows, :] + comm_ref[rows, :]
            ms = jnp.mean(y * y, axis=-1, keepdims=True) + 1e-6
            out_ref[rows, :] = y * lax.rsqrt(ms) * g_ref[:, :]

    return pl.pallas_call(
        body,
        out_shape=jax.ShapeDtypeStruct((HALF, D), jnp.float32),
        in_specs=[
            pl.BlockSpec(memory_space=pltpu.ANY),
            pl.BlockSpec(memory_space=pltpu.VMEM),
        ],
        out_specs=pl.BlockSpec(memory_space=pltpu.VMEM),
        scratch_shapes=[
            pltpu.VMEM((HALF, D), jnp.float32),
            pltpu.VMEM((HALF, D), jnp.float32),
            pltpu.SemaphoreType.DMA,
            pltpu.SemaphoreType.DMA,
            pltpu.SemaphoreType.DMA,
        ],
        compiler_params=pltpu.CompilerParams(collective_id=0),
    )(p, g)


# baseline (device time: 209755 ns/iter reference)
import jax
import jax.numpy as jnp
from jax import lax
from jax.experimental import pallas as pl
from jax.experimental.pallas import tpu as pltpu

HALF = 2048
D = 2048
NCHUNK = 4
CH = HALF // NCHUNK


def kernel(partial, gamma):
    p = partial.reshape(2 * HALF, D)
    g = gamma.reshape(1, D)

    def body(p_ref, g_ref, out_ref, comm_ref, stage_ref,
             copy_sems, send_sem, recv_sem):
        my_x = lax.axis_index("x")
        my_y = lax.axis_index("y")
        my_z = lax.axis_index("z")
        peer = (my_x, 1 - my_y, my_z)

        my_start = my_y * HALF
        peer_start = (1 - my_y) * HALF

        barrier_sem = pltpu.get_barrier_semaphore()
        pl.semaphore_signal(
            barrier_sem, inc=1,
            device_id=peer, device_id_type=pl.DeviceIdType.MESH,
        )
        pl.semaphore_wait(barrier_sem, 1)

        rdma = pltpu.make_async_remote_copy(
            src_ref=p_ref.at[pl.ds(peer_start, HALF), :],
            dst_ref=comm_ref,
            send_sem=send_sem,
            recv_sem=recv_sem,
            device_id=peer,
            device_id_type=pl.DeviceIdType.MESH,
        )
        rdma.start()

        def local_copy(k, slot):
            return pltpu.make_async_copy(
                p_ref.at[pl.ds(my_start + k * CH, CH), :],
                stage_ref.at[slot],
                copy_sems.at[slot],
            )

        local_copy(0, 0).start()
        rdma.wait()

        for k in range(NCHUNK):
            slot = k % 2
            if k + 1 < NCHUNK:
                local_copy(k + 1, (k + 1) % 2).start()
            local_copy(k, slot).wait()
            rows = pl.ds(k * CH, CH)
            y = stage_ref[slot] + comm_ref[rows, :]
            ms = jnp.mean(y * y, axis=-1, keepdims=True) + 1e-6
            out_ref[rows, :] = y * lax.rsqrt(ms) * g_ref[:, :]

    return pl.pallas_call(
        body,
        out_shape=jax.ShapeDtypeStruct((HALF, D), jnp.float32),
        in_specs=[
            pl.BlockSpec(memory_space=pl.ANY),
            pl.BlockSpec(memory_space=pltpu.VMEM),
        ],
        out_specs=pl.BlockSpec(memory_space=pltpu.VMEM),
        scratch_shapes=[
            pltpu.VMEM((HALF, D), jnp.float32),
            pltpu.VMEM((2, CH, D), jnp.float32),
            pltpu.SemaphoreType.DMA((2,)),
            pltpu.SemaphoreType.DMA,
            pltpu.SemaphoreType.DMA,
        ],
        compiler_params=pltpu.CompilerParams(
            collective_id=0,
            vmem_limit_bytes=60 * 1024 * 1024,
        ),
    )(p, g)


# device time: 203836 ns/iter; 1.0290x vs baseline; 1.0290x over previous
import jax
import jax.numpy as jnp
from jax import lax
from jax.experimental import pallas as pl
from jax.experimental.pallas import tpu as pltpu

HALF = 2048
D = 2048
NCHUNK = 8
CH = HALF // NCHUNK


def kernel(partial, gamma):
    p = partial.reshape(2 * HALF, D)
    g = gamma.reshape(1, D)

    def body(p_ref, g_ref, out_ref, comm_ref, stage_ref,
             copy_sems, send_sems, recv_sems):
        my_x = lax.axis_index("x")
        my_y = lax.axis_index("y")
        my_z = lax.axis_index("z")
        peer = (my_x, 1 - my_y, my_z)

        my_start = my_y * HALF
        peer_start = (1 - my_y) * HALF

        barrier_sem = pltpu.get_barrier_semaphore()
        pl.semaphore_signal(
            barrier_sem, inc=1,
            device_id=peer, device_id_type=pl.DeviceIdType.MESH,
        )
        pl.semaphore_wait(barrier_sem, 1)

        rdmas = []
        for i in range(NCHUNK):
            rdmas.append(pltpu.make_async_remote_copy(
                src_ref=p_ref.at[pl.ds(peer_start + i * CH, CH), :],
                dst_ref=comm_ref.at[pl.ds(i * CH, CH), :],
                send_sem=send_sems.at[i],
                recv_sem=recv_sems.at[i],
                device_id=peer,
                device_id_type=pl.DeviceIdType.MESH,
            ))
            rdmas[i].start()

        def local_copy(k, slot):
            return pltpu.make_async_copy(
                p_ref.at[pl.ds(my_start + k * CH, CH), :],
                stage_ref.at[slot],
                copy_sems.at[slot],
            )

        local_copy(0, 0).start()

        for k in range(NCHUNK):
            slot = k % 2
            if k + 1 < NCHUNK:
                local_copy(k + 1, (k + 1) % 2).start()
            local_copy(k, slot).wait()
            rdmas[k].wait_recv()
            rows = pl.ds(k * CH, CH)
            y = stage_ref[slot] + comm_ref[rows, :]
            ms = jnp.mean(y * y, axis=-1, keepdims=True) + 1e-6
            out_ref[rows, :] = y * lax.rsqrt(ms) * g_ref[:, :]

        for k in range(NCHUNK):
            rdmas[k].wait_send()

    return pl.pallas_call(
        body,
        out_shape=jax.ShapeDtypeStruct((HALF, D), jnp.float32),
        in_specs=[
            pl.BlockSpec(memory_space=pl.ANY),
            pl.BlockSpec(memory_space=pltpu.VMEM),
        ],
        out_specs=pl.BlockSpec(memory_space=pltpu.VMEM),
        scratch_shapes=[
            pltpu.VMEM((HALF, D), jnp.float32),
            pltpu.VMEM((2, CH, D), jnp.float32),
            pltpu.SemaphoreType.DMA((2,)),
            pltpu.SemaphoreType.DMA((NCHUNK,)),
            pltpu.SemaphoreType.DMA((NCHUNK,)),
        ],
        compiler_params=pltpu.CompilerParams(
            collective_id=0,
            vmem_limit_bytes=60 * 1024 * 1024,
        ),
    )(p, g)
